# baseline (speedup 1.0000x reference)
import functools

import jax
import jax.numpy as jnp
from jax import lax
from jax.experimental import pallas as pl
from jax.experimental.pallas import tpu as pltpu, tpu_sc as plsc

N = 10000
E = 320000
D = 128
GSZ = 8000
TAIL = N - GSZ

NC, NS = 2, 16
NW = NC * NS
EPW = E // NW
K = 96
KH = K // 2
CHM = EPW // K
TOFF = CHM * K
TK = EPW - TOFF
RPT = 624
ZR = 16
REM = N - NS * RPT

MB = 2000


def _mm_body(notes_ref, w_ref, ei_ref, o_ref, p_ref):
    o_ref[...] = jnp.dot(notes_ref[...], w_ref[...],
                         preferred_element_type=jnp.float32)

    @pl.when(pl.program_id(0) == 0)
    def _():
        p_ref[...] = jnp.bitwise_or(
            jnp.left_shift(ei_ref[0], 16),
            ei_ref[1]).reshape(E // 128, 128)


def _matmul_pack(notes, w, ei):
    return pl.pallas_call(
        _mm_body,
        grid=(N // MB,),
        in_specs=[
            pl.BlockSpec((MB, D), lambda i: (i, 0)),
            pl.BlockSpec((D, D), lambda i: (0, 0)),
            pl.BlockSpec((2, E), lambda i: (0, 0)),
        ],
        out_specs=[
            pl.BlockSpec((MB, D), lambda i: (i, 0)),
            pl.BlockSpec((E // 128, 128), lambda i: (0, 0)),
        ],
        out_shape=[
            jax.ShapeDtypeStruct((N, D), jnp.float32),
            jax.ShapeDtypeStruct((E // 128, 128), jnp.int32),
        ],
    )(notes, w, ei)


def _sc_scatter_fn():
    mesh = plsc.VectorSubcoreMesh(
        core_axis_name="c", subcore_axis_name="s",
        num_cores=NC, num_subcores=NS)

    @functools.partial(
        pl.kernel,
        out_type=jax.ShapeDtypeStruct((NC, N, D), jnp.float32),
        mesh=mesh,
        scratch_types=[
            pltpu.VMEM((K,), jnp.int32),
            pltpu.VMEM((K,), jnp.int32),
            pltpu.VMEM((K,), jnp.int32),
            pltpu.VMEM((K,), jnp.int32),
            pltpu.VMEM((KH,), jnp.int32),
            pltpu.VMEM((KH,), jnp.int32),
            pltpu.VMEM((KH,), jnp.int32),
            pltpu.VMEM((KH,), jnp.int32),
            pltpu.VMEM((KH,), jnp.int32),
            pltpu.VMEM((KH,), jnp.int32),
            pltpu.VMEM((K,), jnp.float32),
            pltpu.VMEM((K,), jnp.float32),
            pltpu.VMEM((TK,), jnp.int32),
            pltpu.VMEM((TK,), jnp.int32),
            pltpu.VMEM((K, D), jnp.float32),
            pltpu.VMEM((K, D), jnp.float32),
            pltpu.VMEM((KH, D), jnp.float32),
            pltpu.VMEM((KH, D), jnp.float32),
            pltpu.VMEM_SHARED((N, D), jnp.float32),
            pltpu.SemaphoreType.DMA,
            pltpu.SemaphoreType.DMA,
            pltpu.SemaphoreType.DMA,
            pltpu.SemaphoreType.DMA,
            pltpu.SemaphoreType.DMA,
            pltpu.SemaphoreType.DMA,
        ],
    )
    def sc_scatter(x0_hbm, packed_hbm, ew_hbm, part_hbm,
                   pc0, pc1, src0, src1,
                   d00, d01, d10, d11, d20, d21,
                   ew0, ew1, src_t, dst_t, rows0, rows1, sw0, sw1,
                   acc, gsem0, gsem1, ssem0, ssem1, psem, zsem):
        c = lax.axis_index("c")
        s = lax.axis_index("s")
        wid = c * NS + s
        ebase = wid * EPW
        pc = (pc0, pc1)
        rows = (rows0, rows1)
        srcb = (src0, src1)
        dstb = ((d00, d01), (d10, d11), (d20, d21))
        ewb = (ew0, ew1)
        sw = (sw0, sw1)
        gsems = (gsem0, gsem1)
        ssems = (ssem0, ssem1)

        def issue_packed(i, r):
            pltpu.async_copy(
                packed_hbm.at[pl.ds(ebase + i * K, K)], pc[r], psem)

        def wait_packed(i, r):
            pltpu.make_async_copy(
                packed_hbm.at[pl.ds(ebase + i * K, K)], pc[r], psem).wait()

        def unpack(b, pr, dr):
            for v in range(K // 16):
                p = pc[pr][pl.ds(v * 16, 16)]
                srcb[b][pl.ds(v * 16, 16)] = jnp.bitwise_and(p, 0xFFFF)
                h, hv = divmod(v, KH // 16)
                dstb[dr][h][pl.ds(hv * 16, 16)] = jnp.right_shift(p, 16)

        def issue_gather(i, b):
            pltpu.async_copy(x0_hbm.at[srcb[b]], rows[b], gsems[b])
            pltpu.async_copy(
                ew_hbm.at[pl.ds(ebase + i * K, K)], ewb[b], gsems[b])

        def wait_gather(i, b):
            pltpu.make_async_copy(
                x0_hbm.at[srcb[b]], rows[b], gsems[b]).wait()
            pltpu.make_async_copy(
                ew_hbm.at[pl.ds(ebase + i * K, K)], ewb[b], gsems[b]).wait()

        def issue_scatter(dr, h):
            pltpu.async_copy(sw[h], acc.at[dstb[dr][h]], ssems[h],
                             add=True)

        def wait_scatter(h):
            pltpu.make_async_copy(
                sw[h], acc.at[dstb[0][h]], ssems[h]).wait()

        def weight_half(b, h):
            @pl.loop(0, KH // 16)
            def _(eb):
                wchunk = ewb[b][pl.ds(h * KH + eb * 16, 16)]
                for l in range(16):
                    wv = jnp.full((16,), 0.0, jnp.float32) + wchunk[l]
                    e = h * KH + eb * 16 + l
                    el = eb * 16 + l
                    for g in range(D // 16):
                        sl = pl.ds(g * 16, 16)
                        sw[h][el, sl] = rows[b][e, sl] * wv

        issue_packed(0, 0)
        issue_packed(1, 1)

        @pl.loop(0, ZR)
        def _(r):
            for g in range(D // 16):
                sw0[r, pl.ds(g * 16, 16)] = jnp.zeros((16,), jnp.float32)

        @pl.loop(0, RPT // ZR)
        def _(j):
            pltpu.async_copy(sw0.at[pl.ds(0, ZR)],
                             acc.at[pl.ds(s * RPT + j * ZR, ZR)], zsem)

        @pl.when(s == NS - 1)
        def _():
            pltpu.async_copy(sw0.at[pl.ds(0, REM)],
                             acc.at[pl.ds(NS * RPT, REM)], zsem)

        @pl.loop(0, RPT // ZR)
        def _(j):
            pltpu.make_async_copy(
                sw0.at[pl.ds(0, ZR)],
                acc.at[pl.ds(s * RPT + j * ZR, ZR)], zsem).wait()

        @pl.when(s == NS - 1)
        def _():
            pltpu.make_async_copy(
                sw0.at[pl.ds(0, REM)],
                acc.at[pl.ds(NS * RPT, REM)], zsem).wait()

        wait_packed(0, 0)
        unpack(0, 0, 0)
        issue_gather(0, 0)

        plsc.subcore_barrier()

        @pl.loop(0, (CHM - 2) // 6)
        def _(j):
            for bi in range(6):
                i = 6 * j + bi
                b = bi % 2
                dr = bi % 3
                drn = (bi + 1) % 3

                wait_packed(i + 1, (bi + 1) % 2)
                unpack(1 - b, (bi + 1) % 2, drn)
                issue_gather(i + 1, 1 - b)
                issue_packed(i + 2, bi % 2)
                wait_gather(i, b)

                @pl.when(i >= 1)
                def _():
                    wait_scatter(0)

                weight_half(b, 0)
                issue_scatter(dr, 0)

                @pl.when(i >= 1)
                def _():
                    wait_scatter(1)

                weight_half(b, 1)
                issue_scatter(dr, 1)

        wait_packed(CHM - 1, 1)
        unpack(1, 1, 1)
        issue_gather(CHM - 1, 1)
        wait_gather(CHM - 2, 0)
        wait_scatter(0)
        weight_half(0, 0)
        issue_scatter(0, 0)
        wait_scatter(1)
        weight_half(0, 1)
        issue_scatter(0, 1)

        wait_gather(CHM - 1, 1)
        wait_scatter(0)
        weight_half(1, 0)
        issue_scatter(1, 0)
        wait_scatter(1)
        weight_half(1, 1)
        issue_scatter(1, 1)

        pltpu.sync_copy(packed_hbm.at[pl.ds(ebase + TOFF, TK)], src_t)
        p_t = src_t[...]
        src_t[...] = jnp.bitwise_and(p_t, 0xFFFF)
        dst_t[...] = jnp.right_shift(p_t, 16)
        pltpu.sync_copy(ew_hbm.at[pl.ds(ebase + TOFF, TK)],
                        ew0.at[pl.ds(0, TK)])
        pltpu.async_copy(x0_hbm.at[src_t], rows0.at[pl.ds(0, TK)],
                         gsem0).wait()
        wait_scatter(0)
        wtail = ew0[pl.ds(0, TK)]
        for l in range(TK):
            wv = jnp.full((16,), 0.0, jnp.float32) + wtail[l]
            for g in range(D // 16):
                sl = pl.ds(g * 16, 16)
                sw0[l, sl] = rows0[l, sl] * wv
        pltpu.sync_copy(sw0.at[pl.ds(0, TK)], acc.at[dst_t], add=True)

        wait_scatter(1)

        plsc.subcore_barrier()

        pltpu.sync_copy(acc.at[pl.ds(s * RPT, RPT)],
                        part_hbm.at[c, pl.ds(s * RPT, RPT)])

        @pl.when(s == NS - 1)
        def _():
            pltpu.sync_copy(acc.at[pl.ds(NS * RPT, REM)],
                            part_hbm.at[c, pl.ds(NS * RPT, REM)])

    return sc_scatter


_sc_scatter = _sc_scatter_fn()


def _merge_body(parts_ref, notes_ref, w_ref, b_ref, o_ref):
    i = pl.program_id(0)

    @pl.when(i < N // MB)
    def _():
        o_ref[...] = jnp.maximum(
            parts_ref[0] + parts_ref[1] + b_ref[...], 0.0)

    @pl.when(i >= N // MB)
    def _():
        o_ref[...] = jnp.dot(notes_ref[...], w_ref[...],
                             preferred_element_type=jnp.float32)


def _merge(parts, notes, w, b):
    nblk = (N + TAIL) // MB
    return pl.pallas_call(
        _merge_body,
        grid=(nblk,),
        in_specs=[
            pl.BlockSpec((NC, MB, D),
                         lambda i: (0, jnp.minimum(i, N // MB - 1), 0)),
            pl.BlockSpec((MB, D),
                         lambda i: (jnp.where(i >= N // MB, GSZ // MB, 0), 0)),
            pl.BlockSpec((D, D), lambda i: (0, 0)),
            pl.BlockSpec((1, D), lambda i: (0, 0)),
        ],
        out_specs=pl.BlockSpec((MB, D), lambda i: (i, 0)),
        out_shape=jax.ShapeDtypeStruct((N + TAIL, D), jnp.float32),
    )(parts, notes, w, b)


def kernel(notes, edge_index, edge_weight, w, b, garment_size):
    del garment_size
    ei = edge_index.astype(jnp.int32)
    x0, packed = _matmul_pack(notes, w, ei)
    parts = _sc_scatter(x0, packed.reshape(E), edge_weight)
    return _merge(parts, notes, w, b.reshape(1, D))


# speedup vs baseline: 1.1786x; 1.1786x over previous
import functools

import jax
import jax.numpy as jnp
from jax import lax
from jax.experimental import pallas as pl
from jax.experimental.pallas import tpu as pltpu, tpu_sc as plsc

N = 10000
E = 320000
D = 128
GSZ = 8000
TAIL = N - GSZ

NC, NS = 2, 16
NW = NC * NS
EPW = E // NW
K = 128
CHM = EPW // K
TOFF = CHM * K
TK = EPW - TOFF
RPT = 624
ZR = 24
REM = N - NS * RPT

MB = 2000


def _mm_body(notes_ref, w_ref, ei_ref, o_ref, p_ref):
    o_ref[...] = jnp.dot(notes_ref[...], w_ref[...],
                         preferred_element_type=jnp.float32)

    @pl.when(pl.program_id(0) == 0)
    def _():
        p_ref[...] = jnp.bitwise_or(
            jnp.left_shift(ei_ref[0], 16),
            ei_ref[1]).reshape(E // 128, 128)


def _matmul_pack(notes, w, ei):
    return pl.pallas_call(
        _mm_body,
        grid=(N // MB,),
        in_specs=[
            pl.BlockSpec((MB, D), lambda i: (i, 0)),
            pl.BlockSpec((D, D), lambda i: (0, 0)),
            pl.BlockSpec((2, E), lambda i: (0, 0)),
        ],
        out_specs=[
            pl.BlockSpec((MB, D), lambda i: (i, 0)),
            pl.BlockSpec((E // 128, 128), lambda i: (0, 0)),
        ],
        out_shape=[
            jax.ShapeDtypeStruct((N, D), jnp.float32),
            jax.ShapeDtypeStruct((E // 128, 128), jnp.int32),
        ],
    )(notes, w, ei)


def _sc_scatter_fn():
    mesh = plsc.VectorSubcoreMesh(
        core_axis_name="c", subcore_axis_name="s",
        num_cores=NC, num_subcores=NS)

    @functools.partial(
        pl.kernel,
        out_type=jax.ShapeDtypeStruct((NC, N, D), jnp.float32),
        mesh=mesh,
        scratch_types=[
            pltpu.VMEM((EPW,), jnp.int32),
            pltpu.VMEM((K,), jnp.int32),
            pltpu.VMEM((K,), jnp.int32),
            pltpu.VMEM((K,), jnp.int32),
            pltpu.VMEM((K,), jnp.int32),
            pltpu.VMEM((K,), jnp.float32),
            pltpu.VMEM((K,), jnp.float32),
            pltpu.VMEM((TK,), jnp.int32),
            pltpu.VMEM((TK,), jnp.int32),
            pltpu.VMEM((TK,), jnp.float32),
            pltpu.VMEM((K, D), jnp.float32),
            pltpu.VMEM((K, D), jnp.float32),
            pltpu.VMEM((ZR, D), jnp.float32),
            pltpu.VMEM_SHARED((N, D), jnp.float32),
            pltpu.SemaphoreType.DMA,
            pltpu.SemaphoreType.DMA,
            pltpu.SemaphoreType.DMA,
            pltpu.SemaphoreType.DMA,
            pltpu.SemaphoreType.DMA,
        ],
    )
    def sc_scatter(x0_hbm, packed_hbm, ew_hbm, part_hbm,
                   packed_v, src0, src1, dst0, dst1, ew0, ew1,
                   src_t, dst_t, ew_t, rows0, rows1, zbuf, acc,
                   gsem0, gsem1, ssem0, ssem1, zsem):
        c = lax.axis_index("c")
        s = lax.axis_index("s")
        wid = c * NS + s
        ebase = wid * EPW
        rows = (rows0, rows1)
        srcb = (src0, src1)
        dstb = (dst0, dst1)
        ewb = (ew0, ew1)
        gsems = (gsem0, gsem1)
        ssems = (ssem0, ssem1)

        pltpu.async_copy(packed_hbm.at[pl.ds(ebase, EPW)], packed_v, gsem0)

        @pl.loop(0, ZR)
        def _(r):
            for g in range(D // 16):
                zbuf[r, pl.ds(g * 16, 16)] = jnp.zeros((16,), jnp.float32)

        @pl.loop(0, RPT // ZR)
        def _(j):
            pltpu.async_copy(zbuf, acc.at[pl.ds(s * RPT + j * ZR, ZR)],
                             zsem)

        @pl.when(s == NS - 1)
        def _():
            pltpu.async_copy(zbuf.at[pl.ds(0, REM)],
                            acc.at[pl.ds(NS * RPT, REM)], zsem)

        @pl.loop(0, RPT // ZR)
        def _(j):
            pltpu.make_async_copy(
                zbuf, acc.at[pl.ds(s * RPT + j * ZR, ZR)], zsem).wait()

        @pl.when(s == NS - 1)
        def _():
            pltpu.make_async_copy(
                zbuf.at[pl.ds(0, REM)],
                acc.at[pl.ds(NS * RPT, REM)], zsem).wait()

        pltpu.make_async_copy(
            packed_hbm.at[pl.ds(ebase, EPW)], packed_v, gsem0).wait()

        def unpack(i, b):
            for v in range(K // 16):
                sl = pl.ds(v * 16, 16)
                p = packed_v[pl.ds(i * K + v * 16, 16)]
                srcb[b][sl] = jnp.bitwise_and(p, 0xFFFF)
                dstb[b][sl] = jnp.right_shift(p, 16)

        def issue_gather(i, b):
            pltpu.async_copy(x0_hbm.at[srcb[b]], rows[b], gsems[b])
            pltpu.async_copy(
                ew_hbm.at[pl.ds(ebase + i * K, K)], ewb[b], gsems[b])

        def wait_gather(i, b):
            pltpu.make_async_copy(
                x0_hbm.at[srcb[b]], rows[b], gsems[b]).wait()
            pltpu.make_async_copy(
                ew_hbm.at[pl.ds(ebase + i * K, K)], ewb[b], gsems[b]).wait()

        def issue_scatter(b):
            pltpu.async_copy(rows[b], acc.at[dstb[b]], ssems[b], add=True)

        def wait_scatter(b):
            pltpu.make_async_copy(
                rows[b], acc.at[dstb[b]], ssems[b]).wait()

        def weight_rows(b):
            @pl.loop(0, K // 16, unroll=4)
            def _(eb):
                wchunk = ewb[b][pl.ds(eb * 16, 16)]
                for l in range(16):
                    wv = jnp.full((16,), 0.0, jnp.float32) + wchunk[l]
                    e = eb * 16 + l
                    for g in range(D // 16):
                        sl = pl.ds(g * 16, 16)
                        rows[b][e, sl] = rows[b][e, sl] * wv

        unpack(0, 0)
        issue_gather(0, 0)
        plsc.subcore_barrier()

        wait_gather(0, 0)
        unpack(1, 1)
        issue_gather(1, 1)
        weight_rows(0)
        issue_scatter(0)

        @pl.loop(0, (CHM - 2) // 2)
        def _(j):
            for bi in range(2):
                i = 1 + 2 * j + bi
                b = (1 + bi) % 2
                wait_gather(i, b)
                wait_scatter(1 - b)
                unpack(i + 1, 1 - b)
                issue_gather(i + 1, 1 - b)
                weight_rows(b)
                issue_scatter(b)

        wait_gather(CHM - 1, 1)
        wait_scatter(0)
        weight_rows(1)
        issue_scatter(1)

        p_t = packed_v[pl.ds(TOFF, TK)]
        src_t[...] = jnp.bitwise_and(p_t, 0xFFFF)
        dst_t[...] = jnp.right_shift(p_t, 16)
        pltpu.sync_copy(ew_hbm.at[pl.ds(ebase + TOFF, TK)], ew_t)
        pltpu.async_copy(x0_hbm.at[src_t], rows0.at[pl.ds(0, TK)],
                         gsem0).wait()
        wtail = ew_t[...]
        for l in range(TK):
            wv = jnp.full((16,), 0.0, jnp.float32) + wtail[l]
            for g in range(D // 16):
                sl = pl.ds(g * 16, 16)
                rows0[l, sl] = rows0[l, sl] * wv
        pltpu.sync_copy(rows0.at[pl.ds(0, TK)], acc.at[dst_t], add=True)

        wait_scatter(1)

        plsc.subcore_barrier()

        pltpu.sync_copy(acc.at[pl.ds(s * RPT, RPT)],
                        part_hbm.at[c, pl.ds(s * RPT, RPT)])

        @pl.when(s == NS - 1)
        def _():
            pltpu.sync_copy(acc.at[pl.ds(NS * RPT, REM)],
                            part_hbm.at[c, pl.ds(NS * RPT, REM)])

    return sc_scatter


_sc_scatter = _sc_scatter_fn()


def _merge_body(parts_ref, notes_ref, w_ref, b_ref, o_ref):
    i = pl.program_id(0)

    @pl.when(i < N // MB)
    def _():
        o_ref[...] = jnp.maximum(
            parts_ref[0] + parts_ref[1] + b_ref[...], 0.0)

    @pl.when(i >= N // MB)
    def _():
        o_ref[...] = jnp.dot(notes_ref[...], w_ref[...],
                             preferred_element_type=jnp.float32)


def _merge(parts, notes, w, b):
    nblk = (N + TAIL) // MB
    return pl.pallas_call(
        _merge_body,
        grid=(nblk,),
        in_specs=[
            pl.BlockSpec((NC, MB, D),
                         lambda i: (0, jnp.minimum(i, N // MB - 1), 0)),
            pl.BlockSpec((MB, D),
                         lambda i: (jnp.where(i >= N // MB, GSZ // MB, 0), 0)),
            pl.BlockSpec((D, D), lambda i: (0, 0)),
            pl.BlockSpec((1, D), lambda i: (0, 0)),
        ],
        out_specs=pl.BlockSpec((MB, D), lambda i: (i, 0)),
        out_shape=jax.ShapeDtypeStruct((N + TAIL, D), jnp.float32),
    )(parts, notes, w, b)


def kernel(notes, edge_index, edge_weight, w, b, garment_size):
    del garment_size
    ei = edge_index.astype(jnp.int32)
    x0, packed = _matmul_pack(notes, w, ei)
    parts = _sc_scatter(x0, packed.reshape(E), edge_weight)
    return _merge(parts, notes, w, b.reshape(1, D))
